# Initial kernel scaffold; baseline (speedup 1.0000x reference)
import jax
import jax.numpy as jnp
from jax.experimental import pallas as pl


def kernel(input, W0, W1, V):
    raise NotImplementedError("write your pallas kernel here")


# speedup vs baseline: 1.3184x; 1.3184x over previous
import functools

import jax
import jax.numpy as jnp
from jax import lax
from jax.experimental import pallas as pl
from jax.experimental.pallas import tpu as pltpu
from jax.experimental.pallas import tpu_sc as plsc

_B = 16384
_F = 26
_EMB = 16
_NW = 32
_SPW = _B // _NW
_C = 64
_NCHUNK = _SPW // _C
_LPC = _C * _F
_NSLICE = _LPC // 128


def _fm_sc_body(idx_hbm, w1_hbm, v_hbm, out_hbm,
                idx_v, rows_v, w1_v, outv_v, sem_v, sem_w):
    wid = lax.axis_index("s") * 2 + lax.axis_index("c")
    lane = lax.broadcasted_iota(jnp.int32, (16,), 0)
    tail_mask = lane < (_F - 16)

    def chunk_body(g, carry):
        base = wid * _SPW + g * _C
        pltpu.sync_copy(idx_hbm.at[pl.ds(base * _F, _LPC)], idx_v)
        copies = []
        for j in range(_NSLICE):
            sl = pl.ds(j * 128, 128)
            copies.append(pltpu.async_copy(v_hbm.at[idx_v.at[sl]],
                                           rows_v.at[sl], sem_v))
            copies.append(pltpu.async_copy(w1_hbm.at[idx_v.at[sl]],
                                           w1_v.at[sl], sem_w))
        for c in copies:
            c.wait()

        def samp_body(i, carry2):
            r0 = i * _F
            x = rows_v[r0, :]
            s = x
            sq = x * x
            for f in range(1, _F):
                x = rows_v[r0 + f, :]
                s = s + x
                sq = sq + x * x
            v1 = w1_v[pl.ds(r0, 16)]
            v2 = jnp.where(tail_mask, w1_v[pl.ds(r0 + 16, 16)], 0.0)
            outv_v[pl.ds(i * _EMB, _EMB)] = (s * s - sq) * 0.5 + v1 + v2
            return carry2

        lax.fori_loop(0, _C, samp_body, 0)
        pltpu.sync_copy(outv_v, out_hbm.at[pl.ds(base * _EMB, _C * _EMB)])
        return carry

    lax.fori_loop(0, _NCHUNK, chunk_body, 0)


def _fm_tc_body(x_ref, w0_ref, o_ref):
    sel = (lax.broadcasted_iota(jnp.int32, (128, 8), 0) // _EMB
           == lax.broadcasted_iota(jnp.int32, (128, 8), 1))
    o_ref[...] = (jnp.dot(x_ref[...], sel.astype(jnp.float32),
                          preferred_element_type=jnp.float32)
                  + w0_ref[0:1, 0:1])


@jax.jit
def _fm(idx_flat, w0_2d, w1_flat, v):
    mesh = plsc.VectorSubcoreMesh(core_axis_name="c", subcore_axis_name="s")
    run = functools.partial(
        pl.kernel,
        out_type=jax.ShapeDtypeStruct((_B * _EMB,), jnp.float32),
        mesh=mesh,
        compiler_params=pltpu.CompilerParams(use_tc_tiling_on_sc=False),
        scratch_types=[
            pltpu.VMEM((_LPC,), jnp.int32),
            pltpu.VMEM((_LPC, _EMB), jnp.float32),
            pltpu.VMEM((_LPC + 16,), jnp.float32),
            pltpu.VMEM((_C * _EMB,), jnp.float32),
            pltpu.SemaphoreType.DMA,
            pltpu.SemaphoreType.DMA,
        ],
    )(_fm_sc_body)
    partial_flat = run(idx_flat, w1_flat, v)
    partial = partial_flat.reshape(_B * _EMB // 128, 128)
    out = pl.pallas_call(
        _fm_tc_body,
        out_shape=jax.ShapeDtypeStruct((_B * _EMB // 128, 8), jnp.float32),
    )(partial, w0_2d)
    return out.reshape(_B, 1)


def kernel(input, W0, W1, V):
    idx_flat = input.reshape(-1).astype(jnp.int32)
    w0_2d = W0.astype(jnp.float32).reshape(1, 1)
    w1_flat = W1.reshape(-1)
    return _fm(idx_flat, w0_2d, w1_flat, V)
